# baseline (speedup 1.0000x reference)
import functools

import jax
import jax.numpy as jnp
from jax import lax
from jax.experimental import pallas as pl
from jax.experimental.pallas import tpu as pltpu
from jax.experimental.pallas import tpu_sc as plsc

_NC = 2
_NS = 16
_NW = _NC * _NS
_L = 16

_RATIO = 0.75


@functools.lru_cache(maxsize=None)
def _build_sc_call(T, B, C):
    R = int(T * (1 - _RATIO))
    G = R * B
    GPW = G // _NW
    ROWS = 64
    NG = GPW // ROWS
    BPW = B // _NW
    NT = T // _L
    assert G % _NW == 0 and GPW % ROWS == 0 and NG % 2 == 0
    assert B % _NW == 0 and T % _L == 0

    mesh = plsc.VectorSubcoreMesh(
        core_axis_name="c", subcore_axis_name="s",
        num_cores=_NC, num_subcores=_NS)

    def body(patches_hbm, fiflat_hbm, fit_hbm,
             kept_hbm, backt_hbm,
             fi_v, idx_v, rows_v, fib_v, back_v, sem0, sem1):
        wid = lax.axis_index("s") * _NC + lax.axis_index("c")
        base = wid * GPW

        pltpu.sync_copy(fiflat_hbm.at[pl.ds(base, GPW)], fi_v)

        def cidx(g, _):
            for j in range(ROWS // _L):
                off = g * ROWS + j * _L
                fi16 = fi_v[pl.ds(off, _L)]
                r = base + off + lax.iota(jnp.int32, _L)
                idx_v[g, pl.ds(j * _L, _L)] = fi16 * B + lax.rem(r, B)
            return 0
        lax.fori_loop(0, NG, cidx, 0)

        def start(g, buf, sem):
            return pltpu.async_copy(
                patches_hbm.at[idx_v.at[g]],
                rows_v.at[pl.ds(buf * ROWS, ROWS)], sem)

        def drain_write(g, buf, sem):
            pltpu.make_async_copy(
                patches_hbm.at[idx_v.at[g]],
                rows_v.at[pl.ds(buf * ROWS, ROWS)], sem).wait()
            pltpu.sync_copy(rows_v.at[pl.ds(buf * ROWS, ROWS)],
                            kept_hbm.at[pl.ds(base + g * ROWS, ROWS)])

        start(0, 0, sem0)

        def gloop(h, _):
            g0 = 2 * h
            g1 = g0 + 1
            start(g1, 1, sem1)
            drain_write(g0, 0, sem0)

            @pl.when(g1 + 1 < NG)
            def _():
                start(g1 + 1, 0, sem0)
            drain_write(g1, 1, sem1)
            return 0
        lax.fori_loop(0, NG // 2, gloop, 0)

        b0 = wid * BPW
        pltpu.sync_copy(fit_hbm.at[pl.ds(b0, BPW)], fib_v)
        lanes = lax.iota(jnp.int32, _L)
        for bl in range(BPW):
            row = jnp.full((_L,), bl, jnp.int32)

            def scat(k, _, row=row, bl=bl):
                j0 = k * _L
                cols = fib_v[bl, pl.ds(j0, _L)]
                plsc.store_scatter(back_v, [row, cols], j0 + lanes)
                return 0
            lax.fori_loop(0, NT, scat, 0)
        pltpu.sync_copy(back_v, backt_hbm.at[pl.ds(b0, BPW)])

    call = pl.kernel(
        body,
        out_type=(jax.ShapeDtypeStruct((T * B, C), jnp.float32),
                  jax.ShapeDtypeStruct((B, T), jnp.int32)),
        mesh=mesh,
        scratch_types=(
            pltpu.VMEM((GPW,), jnp.int32),
            pltpu.VMEM((NG, ROWS), jnp.int32),
            pltpu.VMEM((2 * ROWS, C), jnp.float32),
            pltpu.VMEM((BPW, T), jnp.int32),
            pltpu.VMEM((BPW, T), jnp.int32),
            pltpu.SemaphoreType.DMA,
            pltpu.SemaphoreType.DMA,
        ),
        compiler_params=pltpu.CompilerParams(use_tc_tiling_on_sc=True,
                                             needs_layout_passes=False),
    )
    return call


@functools.lru_cache(maxsize=None)
def _build_tc_fill(T, B, C):
    R = int(T * (1 - _RATIO))
    G = R * B
    F = (T - R) * B
    BLK = 512
    assert F % BLK == 0 and G % BLK == 0
    NBLK = F // BLK
    HEAD = G // BLK

    def body(_kept_any, mask_ref, out_ref):
        out_ref[...] = jnp.broadcast_to(mask_ref[...], (BLK, C))

    return pl.pallas_call(
        body,
        grid=(NBLK,),
        in_specs=[
            pl.BlockSpec(memory_space=pl.ANY),
            pl.BlockSpec((1, C), lambda i: (0, 0)),
        ],
        out_specs=pl.BlockSpec((BLK, C), lambda i: (HEAD + i, 0)),
        out_shape=jax.ShapeDtypeStruct((T * B, C), jnp.float32),
        input_output_aliases={0: 0},
    )


def kernel(patches, forward_indexes, mask_token):
    T, B, C = patches.shape
    sc_call = _build_sc_call(T, B, C)
    tc_fill = _build_tc_fill(T, B, C)
    fi = forward_indexes.astype(jnp.int32)
    patches_flat = patches.reshape(T * B, C)
    fi_flat = fi.reshape(T * B)
    fi_t = fi.T
    kept_head, backward_t = sc_call(patches_flat, fi_flat, fi_t)
    kept_flat = tc_fill(kept_head, mask_token.reshape(1, C))
    return kept_flat.reshape(T, B, C), forward_indexes, backward_t.T


# speedup vs baseline: 2.2698x; 2.2698x over previous
import functools

import jax
import jax.numpy as jnp
from jax import lax
from jax.experimental import pallas as pl
from jax.experimental.pallas import tpu as pltpu
from jax.experimental.pallas import tpu_sc as plsc

_NC = 2
_NS = 16
_NW = _NC * _NS
_L = 16

_RATIO = 0.75
_SC_FILL_ROWS = 0


@functools.lru_cache(maxsize=None)
def _build_alloc(T, B, C):
    def body(out_ref):
        pass

    return pl.pallas_call(
        body,
        out_specs=pl.BlockSpec(memory_space=pl.ANY),
        out_shape=jax.ShapeDtypeStruct((T * B, C), jnp.float32),
    )


@functools.lru_cache(maxsize=None)
def _build_tc_fill(T, B, C):
    R = int(T * (1 - _RATIO))
    G = R * B + _SC_FILL_ROWS
    F = T * B - G
    BLK = 1024
    assert F % BLK == 0
    NBLK = F // BLK

    def body(buf_ref, mask_ref, dummy_ref, fill_v, sem):
        i = pl.program_id(0)

        @pl.when(i == 0)
        def _():
            fill_v[...] = jnp.broadcast_to(mask_ref[...], (BLK, C))
            dummy_ref[...] = jnp.zeros((8, 128), jnp.float32)

        pltpu.make_async_copy(
            fill_v, buf_ref.at[pl.ds(G + i * BLK, BLK)], sem).start()

        @pl.when(i > 0)
        def _():
            pltpu.make_async_copy(
                fill_v, buf_ref.at[pl.ds(G, BLK)], sem).wait()

        @pl.when(i == NBLK - 1)
        def _():
            pltpu.make_async_copy(
                fill_v, buf_ref.at[pl.ds(G, BLK)], sem).wait()

    return pl.pallas_call(
        body,
        grid=(NBLK,),
        in_specs=[
            pl.BlockSpec(memory_space=pl.ANY),
            pl.BlockSpec((1, C), lambda i: (0, 0)),
        ],
        out_specs=pl.BlockSpec((8, 128), lambda i: (0, 0)),
        out_shape=jax.ShapeDtypeStruct((8, 128), jnp.float32),
        scratch_shapes=[
            pltpu.VMEM((BLK, C), jnp.float32),
            pltpu.SemaphoreType.DMA,
        ],
    )


@functools.lru_cache(maxsize=None)
def _build_sc_call(T, B, C):
    R = int(T * (1 - _RATIO))
    G = R * B
    GPW = G // _NW
    ROWS = 64
    NG = GPW // ROWS
    BPW = B // _NW
    NT = T // _L
    FPW = _SC_FILL_ROWS // _NW
    FB = 2 * ROWS
    assert G % _NW == 0 and GPW % ROWS == 0 and NG % 2 == 0
    assert B % _NW == 0 and T % _L == 0
    assert _SC_FILL_ROWS % (_NW * FB) == 0 or _SC_FILL_ROWS == 0
    NF = FPW // FB if FPW else 0

    mesh = plsc.VectorSubcoreMesh(
        core_axis_name="c", subcore_axis_name="s",
        num_cores=_NC, num_subcores=_NS)

    def body(patches_hbm, fiflat_hbm, fit_hbm, fill_hbm, kept_hbm,
             backt_hbm,
             fi_v, idx_v, rows_v, fib_v, back_v, sem0, sem1):
        wid = lax.axis_index("s") * _NC + lax.axis_index("c")
        base = wid * GPW

        pltpu.sync_copy(fiflat_hbm.at[pl.ds(base, GPW)], fi_v)

        def cidx(g, _):
            for j in range(ROWS // _L):
                off = g * ROWS + j * _L
                fi16 = fi_v[pl.ds(off, _L)]
                r = base + off + lax.iota(jnp.int32, _L)
                idx_v[g, pl.ds(j * _L, _L)] = fi16 * B + lax.rem(r, B)
            return 0
        lax.fori_loop(0, NG, cidx, 0)

        def start(g, buf, sem):
            return pltpu.async_copy(
                patches_hbm.at[idx_v.at[g]],
                rows_v.at[pl.ds(buf * ROWS, ROWS)], sem)

        def drain_write(g, buf, sem):
            pltpu.make_async_copy(
                patches_hbm.at[idx_v.at[g]],
                rows_v.at[pl.ds(buf * ROWS, ROWS)], sem).wait()
            pltpu.sync_copy(rows_v.at[pl.ds(buf * ROWS, ROWS)],
                            kept_hbm.at[pl.ds(base + g * ROWS, ROWS)])

        start(0, 0, sem0)

        def gloop(h, _):
            g0 = 2 * h
            g1 = g0 + 1
            start(g1, 1, sem1)
            drain_write(g0, 0, sem0)

            @pl.when(g1 + 1 < NG)
            def _():
                start(g1 + 1, 0, sem0)
            drain_write(g1, 1, sem1)
            return 0
        lax.fori_loop(0, NG // 2, gloop, 0)

        b0 = wid * BPW
        pltpu.sync_copy(fit_hbm.at[pl.ds(b0, BPW)], fib_v)
        lanes = lax.iota(jnp.int32, _L)
        for bl in range(BPW):
            row = jnp.full((_L,), bl, jnp.int32)

            def scat(k, _, row=row, bl=bl):
                j0 = k * _L
                cols = fib_v[bl, pl.ds(j0, _L)]
                plsc.store_scatter(back_v, [row, cols], j0 + lanes)
                return 0
            lax.fori_loop(0, NT, scat, 0)
        pltpu.sync_copy(back_v, backt_hbm.at[pl.ds(b0, BPW)])

        if NF:
            pltpu.sync_copy(fill_hbm, rows_v)
            fbase = G + wid * FPW

            def floop(t, _):
                pltpu.sync_copy(
                    rows_v, kept_hbm.at[pl.ds(fbase + t * FB, FB)])
                return 0
            lax.fori_loop(0, NF, floop, 0)

    call = pl.kernel(
        body,
        out_type=jax.ShapeDtypeStruct((B, T), jnp.int32),
        mesh=mesh,
        scratch_types=(
            pltpu.VMEM((GPW,), jnp.int32),
            pltpu.VMEM((NG, ROWS), jnp.int32),
            pltpu.VMEM((2 * ROWS, C), jnp.float32),
            pltpu.VMEM((BPW, T), jnp.int32),
            pltpu.VMEM((BPW, T), jnp.int32),
            pltpu.SemaphoreType.DMA,
            pltpu.SemaphoreType.DMA,
        ),
        compiler_params=pltpu.CompilerParams(use_tc_tiling_on_sc=True,
                                             needs_layout_passes=False,
                                             has_side_effects=True),
    )
    return call, FB


def kernel(patches, forward_indexes, mask_token):
    T, B, C = patches.shape
    alloc = _build_alloc(T, B, C)
    tc_fill = _build_tc_fill(T, B, C)
    sc_call, fb = _build_sc_call(T, B, C)
    fi = forward_indexes.astype(jnp.int32)
    patches_flat = patches.reshape(T * B, C)
    fi_flat = fi.reshape(T * B)
    fi_t = fi.T
    mask_row = mask_token.reshape(1, C)
    fill_blk = jnp.broadcast_to(mask_row, (fb, C))
    buf = alloc()
    dummy = tc_fill(buf, mask_row)
    backward_t = sc_call(patches_flat, fi_flat, fi_t, fill_blk, buf)
    kept_flat, _, backward_t = lax.optimization_barrier(
        (buf, dummy, backward_t))
    return kept_flat.reshape(T, B, C), forward_indexes, backward_t.T
